# baseline (speedup 1.0000x reference)
import jax
import jax.numpy as jnp
from jax import lax
from jax.experimental import pallas as pl
from jax.experimental.pallas import tpu as pltpu
from jax.experimental.pallas import tpu_sc as plsc

N = 10000
NP = 10240
E = 160000
EP = 163840
B = 64
NW = 32
NBLK = EP // NW // 128
CBLK0 = 64
CBLK1 = 2 * NBLK - CBLK0
RSUB = NP // 16
FC = 128
BN = 1024
GRID_N = NP // BN


def _sc_mesh():
    return plsc.VectorSubcoreMesh(core_axis_name="c", subcore_axis_name="s")


def _SHARES(core, sub):
    return [
        ((core == 0) & (sub < 8), 64 * sub, 64),
        ((core == 0) & (sub >= 8), 512 + 56 * (sub - 8), 56),
        ((core == 1) & (sub < 8), 960 + 24 * sub, 24),
        ((core == 1) & (sub >= 8), 1152 + 16 * (sub - 8), 16),
    ]


def _sc_pass(C, src2d, dst2d, zeros, *ts):
    NB = 2

    def body(src_hbm, dst_hbm, zeros_hbm, *rest):
        t_refs = rest[:C]
        out = rest[C]
        sidx, didx = rest[C + 1:C + 3]
        rows = rest[C + 3:C + 3 + NB]
        gsem = rest[C + 3 + NB:C + 3 + 2 * NB]
        ssem = rest[C + 3 + 2 * NB:C + 3 + 3 * NB]
        acc = rest[C + 3 + 3 * NB]
        core = lax.axis_index("c")
        sub = lax.axis_index("s")

        def load_idx(base, nblk):
            pltpu.sync_copy(src_hbm.at[pl.ds(base, nblk)],
                            sidx.at[pl.ds(0, nblk)])
            pltpu.sync_copy(dst_hbm.at[pl.ds(base, nblk)],
                            didx.at[pl.ds(0, nblk)])

        for cond, base, nblk in _SHARES(core, sub):
            @pl.when(cond)
            def _(base=base, nblk=nblk):
                load_idx(base, nblk)

        def edge_loop(c, nblk):
            def group(g, carry):
                for k in range(NB):
                    jk = g * NB + k

                    @pl.when(g > 0)
                    def _(k=k):
                        pltpu.make_async_copy(
                            rows[k], acc.at[didx.at[0]], ssem[k]).wait()

                    pltpu.async_copy(t_refs[c].at[sidx.at[jk]], rows[k],
                                     gsem[k])
                for k in range(NB):
                    jk = g * NB + k
                    pltpu.make_async_copy(t_refs[c].at[sidx.at[jk]], rows[k],
                                          gsem[k]).wait()
                    pltpu.async_copy(rows[k], acc.at[didx.at[jk]], ssem[k],
                                     add=True)
                return carry

            lax.fori_loop(0, nblk // NB, group, 0)
            for k in range(NB):
                pltpu.make_async_copy(rows[k], acc.at[didx.at[0]],
                                      ssem[k]).wait()

        for c in range(C):
            pltpu.sync_copy(zeros_hbm, acc.at[pl.ds(sub * RSUB, RSUB)])
            plsc.subcore_barrier()

            for cond, base, nblk in _SHARES(core, sub):
                @pl.when(cond)
                def _(c=c, nblk=nblk):
                    edge_loop(c, nblk)

            plsc.subcore_barrier()
            pltpu.sync_copy(acc.at[pl.ds(sub * RSUB, RSUB)],
                            out.at[core, c, pl.ds(sub * RSUB, RSUB)])

    kern = pl.kernel(
        body,
        out_type=jax.ShapeDtypeStruct((2, C, NP, FC), jnp.float32),
        mesh=_sc_mesh(),
        scratch_types=[
            pltpu.VMEM((CBLK0, 128), jnp.int32),
            pltpu.VMEM((CBLK0, 128), jnp.int32),
        ] + [pltpu.VMEM((128, FC), jnp.float32)] * NB +
            [pltpu.SemaphoreType.DMA] * (2 * NB) +
            [pltpu.VMEM_SHARED((NP, FC), jnp.float32)],
    )
    return kern(src2d, dst2d, zeros, *ts)


def _sc_deg(dst2d, ones, zeros):
    def body(dst_hbm, ones_hbm, zeros_hbm, out, didx, ones_v, acc):
        core = lax.axis_index("c")
        sub = lax.axis_index("s")
        w = sub * 2 + core
        pltpu.sync_copy(dst_hbm.at[pl.ds(w * NBLK, NBLK)], didx)
        pltpu.sync_copy(ones_hbm, ones_v)
        pltpu.sync_copy(zeros_hbm, acc.at[pl.ds(sub * RSUB, RSUB)])
        plsc.subcore_barrier()

        def step(j, carry):
            pltpu.sync_copy(ones_v, acc.at[didx.at[j]], add=True)
            return carry

        lax.fori_loop(0, NBLK, step, 0)
        plsc.subcore_barrier()
        pltpu.sync_copy(acc.at[pl.ds(sub * RSUB, RSUB)],
                        out.at[core, pl.ds(sub * RSUB, RSUB)])

    kern = pl.kernel(
        body,
        out_type=jax.ShapeDtypeStruct((2, NP, FC), jnp.float32),
        mesh=_sc_mesh(),
        scratch_types=[
            pltpu.VMEM((NBLK, 128), jnp.int32),
            pltpu.VMEM((128, FC), jnp.float32),
            pltpu.VMEM_SHARED((NP, FC), jnp.float32),
        ],
    )
    return kern(dst2d, ones, zeros)


def _dot(a, b):
    return jnp.dot(a, b, preferred_element_type=jnp.float32)


def _full(shape):
    return pl.BlockSpec(shape, lambda *a: tuple(0 for _ in shape))


def _rows(shape, dim):
    def imap(i, *_):
        return tuple(i if d == dim else 0 for d in range(len(shape)))
    return pl.BlockSpec(shape, imap)


def _tc1(xp, degp, Wv, W1a, W1b):
    def body(x_ref, deg_ref, wv_ref, w1a_ref, w1b_ref, tza_ref, tzb_ref,
             dis_ref):
        deg = deg_ref[0, :, 0:1] + deg_ref[1, :, 0:1] + 1.0
        dis = lax.rsqrt(deg)
        xb = x_ref[...]
        aug = jnp.tanh(_dot(xb, wv_ref[...]))
        z = _dot(aug, w1a_ref[...]) + _dot(xb, w1b_ref[...])
        tza_ref[...] = dis * z[:, :FC]
        tzb_ref[...] = dis * z[:, FC:]
        dis_ref[...] = dis

    return pl.pallas_call(
        body,
        grid=(GRID_N,),
        in_specs=[_rows((BN, 80), 0), _rows((2, BN, FC), 1),
                  _full((80, 80)), _full((80, 256)), _full((80, 256))],
        out_specs=[_rows((BN, FC), 0)] * 2 + [_rows((BN, 1), 0)],
        out_shape=[jax.ShapeDtypeStruct((NP, FC), jnp.float32)] * 2 +
                  [jax.ShapeDtypeStruct((NP, 1), jnp.float32)],
    )(xp, degp, Wv, W1a, W1b)


def _tc2(SA, tZa, tZb, dis, b1r, W2p):
    def body(sa_ref, tza_ref, tzb_ref, dis_ref, b1_ref, w2_ref,
             tz2a_ref, tz2b_ref, ma_ref, mb_ref, mc_ref):
        dis = dis_ref[...]
        z1a = dis * (sa_ref[0, 0] + sa_ref[1, 0] + tza_ref[...])
        z1b = dis * (sa_ref[0, 1] + sa_ref[1, 1] + tzb_ref[...])
        h1 = jnp.maximum(jnp.concatenate([z1a, z1b], axis=1) + b1_ref[...],
                         0.0)
        tz2a_ref[...] = dis * z1a
        tz2b_ref[...] = dis * z1b
        tm1 = dis * _dot(h1, w2_ref[...])
        ma_ref[...] = tm1[:, :FC]
        mb_ref[...] = tm1[:, FC:2 * FC]
        mc_ref[...] = tm1[:, 2 * FC:]

    return pl.pallas_call(
        body,
        grid=(GRID_N,),
        in_specs=[_rows((2, 2, BN, FC), 2)] + [_rows((BN, FC), 0)] * 2 +
                 [_rows((BN, 1), 0), _full((1, 256)), _full((256, 384))],
        out_specs=[_rows((BN, FC), 0)] * 5,
        out_shape=[jax.ShapeDtypeStruct((NP, FC), jnp.float32)] * 5,
    )(SA, tZa, tZb, dis, b1r, W2p)


def _tc3(SB, tZ2a, tZ2b, tM1a, tM1b, tM1c, dis, b1r, b2r, W2p, W3p):
    def body(sb_ref, tz2a_ref, tz2b_ref, ma_ref, mb_ref, mc_ref, dis_ref,
             b1_ref, b2_ref, w2_ref, w3_ref,
             tz3a_ref, tz3b_ref, tya_ref, tyb_ref, tyc_ref,
             tva_ref, tvb_ref, tvc_ref, tvd_ref, tve_ref):
        dis = dis_ref[...]
        z2a = dis * (sb_ref[0, 0] + sb_ref[1, 0] + tz2a_ref[...])
        z2b = dis * (sb_ref[0, 1] + sb_ref[1, 1] + tz2b_ref[...])
        h4 = jnp.maximum(jnp.concatenate([z2a, z2b], axis=1) + b1_ref[...],
                         0.0)
        tz3a_ref[...] = dis * z2a
        tz3b_ref[...] = dis * z2b
        u2a = dis * (sb_ref[0, 2] + sb_ref[1, 2] + ma_ref[...])
        u2b = dis * (sb_ref[0, 3] + sb_ref[1, 3] + mb_ref[...])
        u2c = dis * (sb_ref[0, 4] + sb_ref[1, 4] + mc_ref[...])
        h2 = jnp.maximum(jnp.concatenate([u2a, u2b, u2c], axis=1) +
                         b2_ref[...], 0.0)
        ty = dis * _dot(h4, w2_ref[...])
        tya_ref[...] = ty[:, :FC]
        tyb_ref[...] = ty[:, FC:2 * FC]
        tyc_ref[...] = ty[:, 2 * FC:]
        tv = dis * _dot(h2, w3_ref[...])
        tva_ref[...] = tv[:, :FC]
        tvb_ref[...] = tv[:, FC:2 * FC]
        tvc_ref[...] = tv[:, 2 * FC:3 * FC]
        tvd_ref[...] = tv[:, 3 * FC:4 * FC]
        tve_ref[...] = tv[:, 4 * FC:]

    return pl.pallas_call(
        body,
        grid=(GRID_N,),
        in_specs=[_rows((2, 5, BN, FC), 2)] + [_rows((BN, FC), 0)] * 5 +
                 [_rows((BN, 1), 0), _full((1, 256)), _full((1, 384)),
                  _full((256, 384)), _full((384, 640))],
        out_specs=[_rows((BN, FC), 0)] * 10,
        out_shape=[jax.ShapeDtypeStruct((NP, FC), jnp.float32)] * 10,
    )(SB, tZ2a, tZ2b, tM1a, tM1b, tM1c, dis, b1r, b2r, W2p, W3p)


def _tc4(SCo, tZ3a, tZ3b, tYa, tYb, tYc, tVa, tVb, tVc, tVd, tVe,
         dis, b1r, b3r):
    def body(sc_ref, tz3a_ref, tz3b_ref, tya_ref, tyb_ref, tyc_ref,
             tva_ref, tvb_ref, tvc_ref, tvd_ref, tve_ref, dis_ref,
             b1_ref, b3_ref,
             h6_ref, ty1a_ref, ty1b_ref, ty1c_ref, h3_ref):
        dis = dis_ref[...]
        z3a = dis * (sc_ref[0, 0] + sc_ref[1, 0] + tz3a_ref[...])
        z3b = dis * (sc_ref[0, 1] + sc_ref[1, 1] + tz3b_ref[...])
        h6_ref[...] = jnp.maximum(
            jnp.concatenate([z3a, z3b], axis=1) + b1_ref[...], 0.0)
        y1a = dis * (sc_ref[0, 2] + sc_ref[1, 2] + tya_ref[...])
        y1b = dis * (sc_ref[0, 3] + sc_ref[1, 3] + tyb_ref[...])
        y1c = dis * (sc_ref[0, 4] + sc_ref[1, 4] + tyc_ref[...])
        ty1a_ref[...] = dis * y1a
        ty1b_ref[...] = dis * y1b
        ty1c_ref[...] = dis * y1c
        va = dis * (sc_ref[0, 5] + sc_ref[1, 5] + tva_ref[...])
        vb = dis * (sc_ref[0, 6] + sc_ref[1, 6] + tvb_ref[...])
        vc = dis * (sc_ref[0, 7] + sc_ref[1, 7] + tvc_ref[...])
        vd = dis * (sc_ref[0, 8] + sc_ref[1, 8] + tvd_ref[...])
        ve = dis * (sc_ref[0, 9] + sc_ref[1, 9] + tve_ref[...])
        v = jnp.concatenate([va, vb, vc, vd, ve], axis=1)
        h3_ref[...] = jnp.maximum(v + b3_ref[...], 0.0)

    return pl.pallas_call(
        body,
        grid=(GRID_N,),
        in_specs=[_rows((2, 10, BN, FC), 2)] + [_rows((BN, FC), 0)] * 10 +
                 [_rows((BN, 1), 0), _full((1, 256)), _full((1, 640))],
        out_specs=[_rows((BN, 256), 0)] + [_rows((BN, FC), 0)] * 3 +
                  [_rows((BN, 640), 0)],
        out_shape=[jax.ShapeDtypeStruct((NP, 256), jnp.float32)] +
                  [jax.ShapeDtypeStruct((NP, FC), jnp.float32)] * 3 +
                  [jax.ShapeDtypeStruct((NP, 640), jnp.float32)],
    )(SCo, tZ3a, tZ3b, tYa, tYb, tYc, tVa, tVb, tVc, tVd, tVe, dis, b1r, b3r)


def _tc5(SD, tY1a, tY1b, tY1c, dis, b2r):
    def body(sd_ref, ta_ref, tb_ref, tc_ref, dis_ref, b2_ref, h5_ref):
        dis = dis_ref[...]
        ya = dis * (sd_ref[0, 0] + sd_ref[1, 0] + ta_ref[...])
        yb = dis * (sd_ref[0, 1] + sd_ref[1, 1] + tb_ref[...])
        yc = dis * (sd_ref[0, 2] + sd_ref[1, 2] + tc_ref[...])
        y = jnp.concatenate([ya, yb, yc], axis=1)
        h5_ref[...] = jnp.maximum(y + b2_ref[...], 0.0)

    return pl.pallas_call(
        body,
        grid=(GRID_N,),
        in_specs=[_rows((2, 3, BN, FC), 2)] + [_rows((BN, FC), 0)] * 3 +
                 [_rows((BN, 1), 0), _full((1, 384))],
        out_specs=[_rows((BN, 384), 0)],
        out_shape=[jax.ShapeDtypeStruct((NP, 384), jnp.float32)],
    )(SD, tY1a, tY1b, tY1c, dis, b2r)[0]


def _segmax(batch, h3p, h5p, h6p):
    def body(batch_sm, h3_ref, h5_ref, h6_ref, g3_ref, g5_ref, g6_ref):
        i = pl.program_id(0)

        @pl.when(i == 0)
        def _():
            g3_ref[...] = jnp.full((B, 640), -jnp.inf, jnp.float32)
            g5_ref[...] = jnp.full((B, 384), -jnp.inf, jnp.float32)
            g6_ref[...] = jnp.full((B, 256), -jnp.inf, jnp.float32)

        for r in range(8):
            seg = batch_sm[i * 8 + r]
            g3_ref[pl.ds(seg, 1), :] = jnp.maximum(
                g3_ref[pl.ds(seg, 1), :], h3_ref[r:r + 1, :])
            g5_ref[pl.ds(seg, 1), :] = jnp.maximum(
                g5_ref[pl.ds(seg, 1), :], h5_ref[r:r + 1, :])
            g6_ref[pl.ds(seg, 1), :] = jnp.maximum(
                g6_ref[pl.ds(seg, 1), :], h6_ref[r:r + 1, :])

    grid_spec = pltpu.PrefetchScalarGridSpec(
        num_scalar_prefetch=1,
        grid=(N // 8,),
        in_specs=[pl.BlockSpec((8, 640), lambda i, *_: (i, 0)),
                  pl.BlockSpec((8, 384), lambda i, *_: (i, 0)),
                  pl.BlockSpec((8, 256), lambda i, *_: (i, 0))],
        out_specs=[pl.BlockSpec((B, 640), lambda i, *_: (0, 0)),
                   pl.BlockSpec((B, 384), lambda i, *_: (0, 0)),
                   pl.BlockSpec((B, 256), lambda i, *_: (0, 0))],
    )
    return pl.pallas_call(
        body,
        grid_spec=grid_spec,
        out_shape=[jax.ShapeDtypeStruct((B, 640), jnp.float32),
                   jax.ShapeDtypeStruct((B, 384), jnp.float32),
                   jax.ShapeDtypeStruct((B, 256), jnp.float32)],
        compiler_params=pltpu.CompilerParams(
            dimension_semantics=("arbitrary",)),
    )(batch, h3p, h5p, h6p)


def _head(g3, g5, g6, embp, tgt, Wg1a, Wg1b, Wg1c, bg1r, Wg2p, bg2r,
          Wtp, btr, Wf1a, Wf1b, bf1r, Wf2p, bf2r, Woutp, boutr):
    def body(g3_ref, g5_ref, g6_ref, emb_ref, tgt_ref, wg1a_ref, wg1b_ref,
             wg1c_ref, bg1_ref, wg2_ref, bg2_ref, wt_ref, bt_ref,
             wf1a_ref, wf1b_ref, bf1_ref, wf2_ref, bf2_ref, wo_ref, bo_ref,
             out_ref):
        def fin(a):
            return jnp.where(jnp.isfinite(a), a, 0.0)

        def rowsoftmax(z):
            zm = jnp.max(z, axis=1, keepdims=True)
            e = jnp.exp(z - zm)
            return e / jnp.sum(e, axis=1, keepdims=True)

        g3 = fin(g3_ref[...])
        g5 = fin(g5_ref[...])
        g6 = fin(g6_ref[...])
        g = jnp.maximum(_dot(g3, wg1a_ref[...]) + _dot(g5, wg1b_ref[...]) +
                        _dot(g6, wg1c_ref[...]) + bg1_ref[...], 0.0)
        gg = _dot(g, wg2_ref[...]) + bg2_ref[...]
        gx = rowsoftmax(jnp.tanh(gg)) * gg

        m = jnp.sum(emb_ref[...], axis=1, keepdims=True) * (1.0 / 128.0)
        tgt_v = tgt_ref[...]
        acc = jnp.zeros((B, 1024), jnp.float32)
        for v in range(26):
            acc = acc + jnp.where(tgt_v == v, m[v:v + 1, 0:1], 0.0)
        xt = jnp.maximum(_dot(acc, wt_ref[...]) + bt_ref[...], 0.0)
        xtt = rowsoftmax(jnp.tanh(xt)) * xt

        f1 = jnp.maximum(_dot(gx, wf1a_ref[...]) + _dot(xtt, wf1b_ref[...]) +
                         bf1_ref[...], 0.0)
        f2 = jnp.maximum(_dot(f1, wf2_ref[...]) + bf2_ref[...], 0.0)
        res = _dot(f2, wo_ref[...])
        out_ref[...] = res[:, 0:1] + bo_ref[...]

    return pl.pallas_call(
        body,
        in_specs=[_full((B, 640)), _full((B, 384)), _full((B, 256)),
                  _full((32, 128)), _full((B, 1024)),
                  _full((640, 1024)), _full((384, 1024)), _full((256, 1024)),
                  _full((1, 1024)), _full((1024, 128)), _full((1, 128)),
                  _full((1024, 128)), _full((1, 128)),
                  _full((128, 1024)), _full((128, 1024)), _full((1, 1024)),
                  _full((1024, 512)), _full((1, 512)),
                  _full((512, 128)), _full((1, 1))],
        out_specs=[_full((B, 1))],
        out_shape=[jax.ShapeDtypeStruct((B, 1), jnp.float32)],
    )(g3, g5, g6, embp, tgt, Wg1a, Wg1b, Wg1c, bg1r, Wg2p, bg2r,
      Wtp, btr, Wf1a, Wf1b, bf1r, Wf2p, bf2r, Woutp, boutr)[0]


def kernel(x, edge_index, batch, target, W_vae, W1, b1, W2, b2, W3, b3,
           Wg1, bg1, Wg2, bg2, emb_xt, W_tran, b_tran, Wf1, bf1, Wf2, bf2,
           Wout, bout):
    f32 = jnp.float32

    def pad2(a, r, c):
        return jnp.zeros((r, c), f32).at[:a.shape[0], :a.shape[1]].set(a)

    xp = pad2(x, NP, 80)
    Wv = pad2(W_vae, 80, 80)
    W1a = pad2(W1[:78], 80, 256)
    W1b = pad2(W1[78:], 80, 256)
    W2p = pad2(W2, 256, 384)
    W3p = pad2(W3, 384, 640)
    b1r = pad2(b1[None], 1, 256)
    b2r = pad2(b2[None], 1, 384)
    b3r = pad2(b3[None], 1, 640)
    Wg1a = pad2(Wg1[:624], 640, 1024)
    Wg1b = pad2(Wg1[624:936], 384, 1024)
    Wg1c = pad2(Wg1[936:], 256, 1024)
    Wtp = pad2(W_tran, 1024, 128)
    Woutp = pad2(Wout, 512, 128)
    embp = pad2(emb_xt, 32, 128)
    tgt = jnp.full((B, 1024), 26, jnp.int32).at[:, :1000].set(target)

    src2d = jnp.zeros((EP,), jnp.int32).at[:E].set(edge_index[0]).reshape(
        EP // 128, 128)
    dst2d = jnp.full((EP,), N, jnp.int32).at[:E].set(edge_index[1]).reshape(
        EP // 128, 128)
    zeros128 = jnp.zeros((RSUB, FC), f32)
    ones128 = jnp.ones((128, FC), f32)

    degp = _sc_deg(dst2d, ones128, zeros128)
    tZa, tZb, dis = _tc1(xp, degp, Wv, W1a, W1b)
    SA = _sc_pass(2, src2d, dst2d, zeros128, tZa, tZb)
    tZ2a, tZ2b, tM1a, tM1b, tM1c = _tc2(SA, tZa, tZb, dis, b1r, W2p)
    SB = _sc_pass(5, src2d, dst2d, zeros128, tZ2a, tZ2b, tM1a, tM1b, tM1c)
    tZ3a, tZ3b, tYa, tYb, tYc, tVa, tVb, tVc, tVd, tVe = _tc3(
        SB, tZ2a, tZ2b, tM1a, tM1b, tM1c, dis, b1r, b2r, W2p, W3p)
    SCo = _sc_pass(10, src2d, dst2d, zeros128,
                   tZ3a, tZ3b, tYa, tYb, tYc, tVa, tVb, tVc, tVd, tVe)
    h6p, tY1a, tY1b, tY1c, h3p = _tc4(
        SCo, tZ3a, tZ3b, tYa, tYb, tYc, tVa, tVb, tVc, tVd, tVe,
        dis, b1r, b3r)
    SD = _sc_pass(3, src2d, dst2d, zeros128, tY1a, tY1b, tY1c)
    h5p = _tc5(SD, tY1a, tY1b, tY1c, dis, b2r)
    g3, g5, g6 = _segmax(batch, h3p, h5p, h6p)
    return _head(g3, g5, g6, embp, tgt, Wg1a, Wg1b, Wg1c, bg1[None], Wg2,
                 bg2[None], Wtp, b_tran[None], Wf1[:128], Wf1[128:],
                 bf1[None], Wf2, bf2[None], Woutp, bout[None])


# speedup vs baseline: 1.1381x; 1.1381x over previous
import jax
import jax.numpy as jnp
from jax import lax
from jax.experimental import pallas as pl
from jax.experimental.pallas import tpu as pltpu
from jax.experimental.pallas import tpu_sc as plsc

N = 10000
NP = 10240
E = 160000
EP = 163840
B = 64
NW = 32
NBLK = EP // NW // 128
CBLK0 = 64
CBLK1 = 2 * NBLK - CBLK0
RSUB = NP // 16
FC = 128
BN = 1024
GRID_N = NP // BN


def _sc_mesh():
    return plsc.VectorSubcoreMesh(core_axis_name="c", subcore_axis_name="s")


def _SHARES(core, sub):
    return [
        (core == 0, sub * (2 * NBLK), CBLK0),
        (core == 1, sub * (2 * NBLK) + CBLK0, CBLK1),
    ]


def _sc_pass(C, src2d, dst2d, zeros, *ts):
    NB = 2

    def body(src_hbm, dst_hbm, zeros_hbm, *rest):
        t_refs = rest[:C]
        out = rest[C]
        sidx, didx = rest[C + 1:C + 3]
        rows = rest[C + 3:C + 3 + NB]
        gsem = rest[C + 3 + NB:C + 3 + 2 * NB]
        ssem = rest[C + 3 + 2 * NB:C + 3 + 3 * NB]
        acc = rest[C + 3 + 3 * NB]
        core = lax.axis_index("c")
        sub = lax.axis_index("s")

        def load_idx(base, nblk):
            pltpu.sync_copy(src_hbm.at[pl.ds(base, nblk)],
                            sidx.at[pl.ds(0, nblk)])
            pltpu.sync_copy(dst_hbm.at[pl.ds(base, nblk)],
                            didx.at[pl.ds(0, nblk)])

        for cond, base, nblk in _SHARES(core, sub):
            @pl.when(cond)
            def _(base=base, nblk=nblk):
                load_idx(base, nblk)

        def edge_loop(c, nblk):
            def group(g, carry):
                for k in range(NB):
                    jk = g * NB + k

                    @pl.when(g > 0)
                    def _(k=k):
                        pltpu.make_async_copy(
                            rows[k], acc.at[didx.at[0]], ssem[k]).wait()

                    pltpu.async_copy(t_refs[c].at[sidx.at[jk]], rows[k],
                                     gsem[k])
                for k in range(NB):
                    jk = g * NB + k
                    pltpu.make_async_copy(t_refs[c].at[sidx.at[jk]], rows[k],
                                          gsem[k]).wait()
                    pltpu.async_copy(rows[k], acc.at[didx.at[jk]], ssem[k],
                                     add=True)
                return carry

            lax.fori_loop(0, nblk // NB, group, 0)
            for k in range(NB):
                pltpu.make_async_copy(rows[k], acc.at[didx.at[0]],
                                      ssem[k]).wait()

        for c in range(C):
            pltpu.sync_copy(zeros_hbm, acc.at[pl.ds(sub * RSUB, RSUB)])
            plsc.subcore_barrier()

            for cond, base, nblk in _SHARES(core, sub):
                @pl.when(cond)
                def _(c=c, nblk=nblk):
                    edge_loop(c, nblk)

            plsc.subcore_barrier()
            pltpu.sync_copy(acc.at[pl.ds(sub * RSUB, RSUB)],
                            out.at[core, c, pl.ds(sub * RSUB, RSUB)])

    kern = pl.kernel(
        body,
        out_type=jax.ShapeDtypeStruct((2, C, NP, FC), jnp.float32),
        mesh=_sc_mesh(),
        scratch_types=[
            pltpu.VMEM((CBLK0, 128), jnp.int32),
            pltpu.VMEM((CBLK0, 128), jnp.int32),
        ] + [pltpu.VMEM((128, FC), jnp.float32)] * NB +
            [pltpu.SemaphoreType.DMA] * (2 * NB) +
            [pltpu.VMEM_SHARED((NP, FC), jnp.float32)],
    )
    return kern(src2d, dst2d, zeros, *ts)


def _sc_deg(dst2d, ones, zeros):
    def body(dst_hbm, ones_hbm, zeros_hbm, out, didx, ones_v, acc):
        core = lax.axis_index("c")
        sub = lax.axis_index("s")
        w = sub * 2 + core
        pltpu.sync_copy(dst_hbm.at[pl.ds(w * NBLK, NBLK)], didx)
        pltpu.sync_copy(ones_hbm, ones_v)
        pltpu.sync_copy(zeros_hbm, acc.at[pl.ds(sub * RSUB, RSUB)])
        plsc.subcore_barrier()

        def step(j, carry):
            pltpu.sync_copy(ones_v, acc.at[didx.at[j]], add=True)
            return carry

        lax.fori_loop(0, NBLK, step, 0)
        plsc.subcore_barrier()
        pltpu.sync_copy(acc.at[pl.ds(sub * RSUB, RSUB)],
                        out.at[core, pl.ds(sub * RSUB, RSUB)])

    kern = pl.kernel(
        body,
        out_type=jax.ShapeDtypeStruct((2, NP, FC), jnp.float32),
        mesh=_sc_mesh(),
        scratch_types=[
            pltpu.VMEM((NBLK, 128), jnp.int32),
            pltpu.VMEM((128, FC), jnp.float32),
            pltpu.VMEM_SHARED((NP, FC), jnp.float32),
        ],
    )
    return kern(dst2d, ones, zeros)


def _dot(a, b):
    return jnp.dot(a, b, preferred_element_type=jnp.float32)


def _full(shape):
    return pl.BlockSpec(shape, lambda *a: tuple(0 for _ in shape))


def _rows(shape, dim):
    def imap(i, *_):
        return tuple(i if d == dim else 0 for d in range(len(shape)))
    return pl.BlockSpec(shape, imap)


def _tc1(xp, degp, Wv, W1a, W1b):
    def body(x_ref, deg_ref, wv_ref, w1a_ref, w1b_ref, tza_ref, tzb_ref,
             dis_ref):
        deg = deg_ref[0, :, 0:1] + deg_ref[1, :, 0:1] + 1.0
        dis = lax.rsqrt(deg)
        xb = x_ref[...]
        aug = jnp.tanh(_dot(xb, wv_ref[...]))
        z = _dot(aug, w1a_ref[...]) + _dot(xb, w1b_ref[...])
        tza_ref[...] = dis * z[:, :FC]
        tzb_ref[...] = dis * z[:, FC:]
        dis_ref[...] = dis

    return pl.pallas_call(
        body,
        grid=(GRID_N,),
        in_specs=[_rows((BN, 80), 0), _rows((2, BN, FC), 1),
                  _full((80, 80)), _full((80, 256)), _full((80, 256))],
        out_specs=[_rows((BN, FC), 0)] * 2 + [_rows((BN, 1), 0)],
        out_shape=[jax.ShapeDtypeStruct((NP, FC), jnp.float32)] * 2 +
                  [jax.ShapeDtypeStruct((NP, 1), jnp.float32)],
    )(xp, degp, Wv, W1a, W1b)


def _tc2(SA, tZa, tZb, dis, b1r, W2p):
    def body(sa_ref, tza_ref, tzb_ref, dis_ref, b1_ref, w2_ref,
             tz2a_ref, tz2b_ref, ma_ref, mb_ref, mc_ref):
        dis = dis_ref[...]
        z1a = dis * (sa_ref[0, 0] + sa_ref[1, 0] + tza_ref[...])
        z1b = dis * (sa_ref[0, 1] + sa_ref[1, 1] + tzb_ref[...])
        h1 = jnp.maximum(jnp.concatenate([z1a, z1b], axis=1) + b1_ref[...],
                         0.0)
        tz2a_ref[...] = dis * z1a
        tz2b_ref[...] = dis * z1b
        tm1 = dis * _dot(h1, w2_ref[...])
        ma_ref[...] = tm1[:, :FC]
        mb_ref[...] = tm1[:, FC:2 * FC]
        mc_ref[...] = tm1[:, 2 * FC:]

    return pl.pallas_call(
        body,
        grid=(GRID_N,),
        in_specs=[_rows((2, 2, BN, FC), 2)] + [_rows((BN, FC), 0)] * 2 +
                 [_rows((BN, 1), 0), _full((1, 256)), _full((256, 384))],
        out_specs=[_rows((BN, FC), 0)] * 5,
        out_shape=[jax.ShapeDtypeStruct((NP, FC), jnp.float32)] * 5,
    )(SA, tZa, tZb, dis, b1r, W2p)


def _tc3(SB, tZ2a, tZ2b, tM1a, tM1b, tM1c, dis, b1r, b2r, W2p, W3p):
    def body(sb_ref, tz2a_ref, tz2b_ref, ma_ref, mb_ref, mc_ref, dis_ref,
             b1_ref, b2_ref, w2_ref, w3_ref,
             tz3a_ref, tz3b_ref, tya_ref, tyb_ref, tyc_ref,
             tva_ref, tvb_ref, tvc_ref, tvd_ref, tve_ref):
        dis = dis_ref[...]
        z2a = dis * (sb_ref[0, 0] + sb_ref[1, 0] + tz2a_ref[...])
        z2b = dis * (sb_ref[0, 1] + sb_ref[1, 1] + tz2b_ref[...])
        h4 = jnp.maximum(jnp.concatenate([z2a, z2b], axis=1) + b1_ref[...],
                         0.0)
        tz3a_ref[...] = dis * z2a
        tz3b_ref[...] = dis * z2b
        u2a = dis * (sb_ref[0, 2] + sb_ref[1, 2] + ma_ref[...])
        u2b = dis * (sb_ref[0, 3] + sb_ref[1, 3] + mb_ref[...])
        u2c = dis * (sb_ref[0, 4] + sb_ref[1, 4] + mc_ref[...])
        h2 = jnp.maximum(jnp.concatenate([u2a, u2b, u2c], axis=1) +
                         b2_ref[...], 0.0)
        ty = dis * _dot(h4, w2_ref[...])
        tya_ref[...] = ty[:, :FC]
        tyb_ref[...] = ty[:, FC:2 * FC]
        tyc_ref[...] = ty[:, 2 * FC:]
        tv = dis * _dot(h2, w3_ref[...])
        tva_ref[...] = tv[:, :FC]
        tvb_ref[...] = tv[:, FC:2 * FC]
        tvc_ref[...] = tv[:, 2 * FC:3 * FC]
        tvd_ref[...] = tv[:, 3 * FC:4 * FC]
        tve_ref[...] = tv[:, 4 * FC:]

    return pl.pallas_call(
        body,
        grid=(GRID_N,),
        in_specs=[_rows((2, 5, BN, FC), 2)] + [_rows((BN, FC), 0)] * 5 +
                 [_rows((BN, 1), 0), _full((1, 256)), _full((1, 384)),
                  _full((256, 384)), _full((384, 640))],
        out_specs=[_rows((BN, FC), 0)] * 10,
        out_shape=[jax.ShapeDtypeStruct((NP, FC), jnp.float32)] * 10,
    )(SB, tZ2a, tZ2b, tM1a, tM1b, tM1c, dis, b1r, b2r, W2p, W3p)


def _tc4(SCo, tZ3a, tZ3b, tYa, tYb, tYc, tVa, tVb, tVc, tVd, tVe,
         dis, b1r, b3r):
    def body(sc_ref, tz3a_ref, tz3b_ref, tya_ref, tyb_ref, tyc_ref,
             tva_ref, tvb_ref, tvc_ref, tvd_ref, tve_ref, dis_ref,
             b1_ref, b3_ref,
             h6_ref, ty1a_ref, ty1b_ref, ty1c_ref, h3_ref):
        dis = dis_ref[...]
        z3a = dis * (sc_ref[0, 0] + sc_ref[1, 0] + tz3a_ref[...])
        z3b = dis * (sc_ref[0, 1] + sc_ref[1, 1] + tz3b_ref[...])
        h6_ref[...] = jnp.maximum(
            jnp.concatenate([z3a, z3b], axis=1) + b1_ref[...], 0.0)
        y1a = dis * (sc_ref[0, 2] + sc_ref[1, 2] + tya_ref[...])
        y1b = dis * (sc_ref[0, 3] + sc_ref[1, 3] + tyb_ref[...])
        y1c = dis * (sc_ref[0, 4] + sc_ref[1, 4] + tyc_ref[...])
        ty1a_ref[...] = dis * y1a
        ty1b_ref[...] = dis * y1b
        ty1c_ref[...] = dis * y1c
        va = dis * (sc_ref[0, 5] + sc_ref[1, 5] + tva_ref[...])
        vb = dis * (sc_ref[0, 6] + sc_ref[1, 6] + tvb_ref[...])
        vc = dis * (sc_ref[0, 7] + sc_ref[1, 7] + tvc_ref[...])
        vd = dis * (sc_ref[0, 8] + sc_ref[1, 8] + tvd_ref[...])
        ve = dis * (sc_ref[0, 9] + sc_ref[1, 9] + tve_ref[...])
        v = jnp.concatenate([va, vb, vc, vd, ve], axis=1)
        h3_ref[...] = jnp.maximum(v + b3_ref[...], 0.0)

    return pl.pallas_call(
        body,
        grid=(GRID_N,),
        in_specs=[_rows((2, 10, BN, FC), 2)] + [_rows((BN, FC), 0)] * 10 +
                 [_rows((BN, 1), 0), _full((1, 256)), _full((1, 640))],
        out_specs=[_rows((BN, 256), 0)] + [_rows((BN, FC), 0)] * 3 +
                  [_rows((BN, 640), 0)],
        out_shape=[jax.ShapeDtypeStruct((NP, 256), jnp.float32)] +
                  [jax.ShapeDtypeStruct((NP, FC), jnp.float32)] * 3 +
                  [jax.ShapeDtypeStruct((NP, 640), jnp.float32)],
    )(SCo, tZ3a, tZ3b, tYa, tYb, tYc, tVa, tVb, tVc, tVd, tVe, dis, b1r, b3r)


def _tc5(SD, tY1a, tY1b, tY1c, dis, b2r):
    def body(sd_ref, ta_ref, tb_ref, tc_ref, dis_ref, b2_ref, h5_ref):
        dis = dis_ref[...]
        ya = dis * (sd_ref[0, 0] + sd_ref[1, 0] + ta_ref[...])
        yb = dis * (sd_ref[0, 1] + sd_ref[1, 1] + tb_ref[...])
        yc = dis * (sd_ref[0, 2] + sd_ref[1, 2] + tc_ref[...])
        y = jnp.concatenate([ya, yb, yc], axis=1)
        h5_ref[...] = jnp.maximum(y + b2_ref[...], 0.0)

    return pl.pallas_call(
        body,
        grid=(GRID_N,),
        in_specs=[_rows((2, 3, BN, FC), 2)] + [_rows((BN, FC), 0)] * 3 +
                 [_rows((BN, 1), 0), _full((1, 384))],
        out_specs=[_rows((BN, 384), 0)],
        out_shape=[jax.ShapeDtypeStruct((NP, 384), jnp.float32)],
    )(SD, tY1a, tY1b, tY1c, dis, b2r)[0]


def _segmax(batch, h3p, h5p, h6p):
    def body(batch_sm, h3_ref, h5_ref, h6_ref, g3_ref, g5_ref, g6_ref):
        i = pl.program_id(0)

        @pl.when(i == 0)
        def _():
            g3_ref[...] = jnp.full((B, 640), -jnp.inf, jnp.float32)
            g5_ref[...] = jnp.full((B, 384), -jnp.inf, jnp.float32)
            g6_ref[...] = jnp.full((B, 256), -jnp.inf, jnp.float32)

        for r in range(8):
            seg = batch_sm[i * 8 + r]
            g3_ref[pl.ds(seg, 1), :] = jnp.maximum(
                g3_ref[pl.ds(seg, 1), :], h3_ref[r:r + 1, :])
            g5_ref[pl.ds(seg, 1), :] = jnp.maximum(
                g5_ref[pl.ds(seg, 1), :], h5_ref[r:r + 1, :])
            g6_ref[pl.ds(seg, 1), :] = jnp.maximum(
                g6_ref[pl.ds(seg, 1), :], h6_ref[r:r + 1, :])

    grid_spec = pltpu.PrefetchScalarGridSpec(
        num_scalar_prefetch=1,
        grid=(N // 8,),
        in_specs=[pl.BlockSpec((8, 640), lambda i, *_: (i, 0)),
                  pl.BlockSpec((8, 384), lambda i, *_: (i, 0)),
                  pl.BlockSpec((8, 256), lambda i, *_: (i, 0))],
        out_specs=[pl.BlockSpec((B, 640), lambda i, *_: (0, 0)),
                   pl.BlockSpec((B, 384), lambda i, *_: (0, 0)),
                   pl.BlockSpec((B, 256), lambda i, *_: (0, 0))],
    )
    return pl.pallas_call(
        body,
        grid_spec=grid_spec,
        out_shape=[jax.ShapeDtypeStruct((B, 640), jnp.float32),
                   jax.ShapeDtypeStruct((B, 384), jnp.float32),
                   jax.ShapeDtypeStruct((B, 256), jnp.float32)],
        compiler_params=pltpu.CompilerParams(
            dimension_semantics=("arbitrary",)),
    )(batch, h3p, h5p, h6p)


def _head(g3, g5, g6, embp, tgt, Wg1a, Wg1b, Wg1c, bg1r, Wg2p, bg2r,
          Wtp, btr, Wf1a, Wf1b, bf1r, Wf2p, bf2r, Woutp, boutr):
    def body(g3_ref, g5_ref, g6_ref, emb_ref, tgt_ref, wg1a_ref, wg1b_ref,
             wg1c_ref, bg1_ref, wg2_ref, bg2_ref, wt_ref, bt_ref,
             wf1a_ref, wf1b_ref, bf1_ref, wf2_ref, bf2_ref, wo_ref, bo_ref,
             out_ref):
        def fin(a):
            return jnp.where(jnp.isfinite(a), a, 0.0)

        def rowsoftmax(z):
            zm = jnp.max(z, axis=1, keepdims=True)
            e = jnp.exp(z - zm)
            return e / jnp.sum(e, axis=1, keepdims=True)

        g3 = fin(g3_ref[...])
        g5 = fin(g5_ref[...])
        g6 = fin(g6_ref[...])
        g = jnp.maximum(_dot(g3, wg1a_ref[...]) + _dot(g5, wg1b_ref[...]) +
                        _dot(g6, wg1c_ref[...]) + bg1_ref[...], 0.0)
        gg = _dot(g, wg2_ref[...]) + bg2_ref[...]
        gx = rowsoftmax(jnp.tanh(gg)) * gg

        m = jnp.sum(emb_ref[...], axis=1, keepdims=True) * (1.0 / 128.0)
        tgt_v = tgt_ref[...]
        acc = jnp.zeros((B, 1024), jnp.float32)
        for v in range(26):
            acc = acc + jnp.where(tgt_v == v, m[v:v + 1, 0:1], 0.0)
        xt = jnp.maximum(_dot(acc, wt_ref[...]) + bt_ref[...], 0.0)
        xtt = rowsoftmax(jnp.tanh(xt)) * xt

        f1 = jnp.maximum(_dot(gx, wf1a_ref[...]) + _dot(xtt, wf1b_ref[...]) +
                         bf1_ref[...], 0.0)
        f2 = jnp.maximum(_dot(f1, wf2_ref[...]) + bf2_ref[...], 0.0)
        res = _dot(f2, wo_ref[...])
        out_ref[...] = res[:, 0:1] + bo_ref[...]

    return pl.pallas_call(
        body,
        in_specs=[_full((B, 640)), _full((B, 384)), _full((B, 256)),
                  _full((32, 128)), _full((B, 1024)),
                  _full((640, 1024)), _full((384, 1024)), _full((256, 1024)),
                  _full((1, 1024)), _full((1024, 128)), _full((1, 128)),
                  _full((1024, 128)), _full((1, 128)),
                  _full((128, 1024)), _full((128, 1024)), _full((1, 1024)),
                  _full((1024, 512)), _full((1, 512)),
                  _full((512, 128)), _full((1, 1))],
        out_specs=[_full((B, 1))],
        out_shape=[jax.ShapeDtypeStruct((B, 1), jnp.float32)],
    )(g3, g5, g6, embp, tgt, Wg1a, Wg1b, Wg1c, bg1r, Wg2p, bg2r,
      Wtp, btr, Wf1a, Wf1b, bf1r, Wf2p, bf2r, Woutp, boutr)[0]


def kernel(x, edge_index, batch, target, W_vae, W1, b1, W2, b2, W3, b3,
           Wg1, bg1, Wg2, bg2, emb_xt, W_tran, b_tran, Wf1, bf1, Wf2, bf2,
           Wout, bout):
    f32 = jnp.float32

    def pad2(a, r, c):
        return jnp.zeros((r, c), f32).at[:a.shape[0], :a.shape[1]].set(a)

    xp = pad2(x, NP, 80)
    Wv = pad2(W_vae, 80, 80)
    W1a = pad2(W1[:78], 80, 256)
    W1b = pad2(W1[78:], 80, 256)
    W2p = pad2(W2, 256, 384)
    W3p = pad2(W3, 384, 640)
    b1r = pad2(b1[None], 1, 256)
    b2r = pad2(b2[None], 1, 384)
    b3r = pad2(b3[None], 1, 640)
    Wg1a = pad2(Wg1[:624], 640, 1024)
    Wg1b = pad2(Wg1[624:936], 384, 1024)
    Wg1c = pad2(Wg1[936:], 256, 1024)
    Wtp = pad2(W_tran, 1024, 128)
    Woutp = pad2(Wout, 512, 128)
    embp = pad2(emb_xt, 32, 128)
    tgt = jnp.full((B, 1024), 26, jnp.int32).at[:, :1000].set(target)

    src2d = jnp.zeros((EP,), jnp.int32).at[:E].set(edge_index[0]).reshape(
        EP // 128, 128)
    dst2d = jnp.full((EP,), N, jnp.int32).at[:E].set(edge_index[1]).reshape(
        EP // 128, 128)
    zeros128 = jnp.zeros((RSUB, FC), f32)
    ones128 = jnp.ones((128, FC), f32)

    degp = _sc_deg(dst2d, ones128, zeros128)
    tZa, tZb, dis = _tc1(xp, degp, Wv, W1a, W1b)
    SA = _sc_pass(2, src2d, dst2d, zeros128, tZa, tZb)
    tZ2a, tZ2b, tM1a, tM1b, tM1c = _tc2(SA, tZa, tZb, dis, b1r, W2p)
    SB = _sc_pass(5, src2d, dst2d, zeros128, tZ2a, tZ2b, tM1a, tM1b, tM1c)
    tZ3a, tZ3b, tYa, tYb, tYc, tVa, tVb, tVc, tVd, tVe = _tc3(
        SB, tZ2a, tZ2b, tM1a, tM1b, tM1c, dis, b1r, b2r, W2p, W3p)
    SCo = _sc_pass(10, src2d, dst2d, zeros128,
                   tZ3a, tZ3b, tYa, tYb, tYc, tVa, tVb, tVc, tVd, tVe)
    h6p, tY1a, tY1b, tY1c, h3p = _tc4(
        SCo, tZ3a, tZ3b, tYa, tYb, tYc, tVa, tVb, tVc, tVd, tVe,
        dis, b1r, b3r)
    SD = _sc_pass(3, src2d, dst2d, zeros128, tY1a, tY1b, tY1c)
    h5p = _tc5(SD, tY1a, tY1b, tY1c, dis, b2r)
    g3, g5, g6 = _segmax(batch, h3p, h5p, h6p)
    return _head(g3, g5, g6, embp, tgt, Wg1a, Wg1b, Wg1c, bg1[None], Wg2,
                 bg2[None], Wtp, b_tran[None], Wf1[:128], Wf1[128:],
                 bf1[None], Wf2, bf2[None], Woutp, bout[None])
